# baseline (speedup 1.0000x reference)
import functools

import jax
import jax.numpy as jnp
from jax import lax
from jax.experimental import pallas as pl
from jax.experimental.pallas import tpu as pltpu
from jax.experimental.pallas import tpu_sc as plsc

MARGIN = 0.4
NUM_CLASSES = 64
BATCH = 4096
DIM = 256

NC = 2
NS = 16
NW = NC * NS
ROWS = BATCH // NW
NBLK = DIM // 16


def _rsqrt16(x):
    xi = plsc.bitcast(x, jnp.int32)
    yi = jnp.full((16,), 0x5F3759DF, jnp.int32) - (xi >> 1)
    y = plsc.bitcast(yi, jnp.float32)
    for _ in range(3):
        y = y * (1.5 - 0.5 * x * y * y)
    return y


def _sc_body(features_hbm, labels_hbm, out_hbm, feats_v, labels_v, sums_v):
    wid = lax.axis_index("s") * NC + lax.axis_index("c")
    base = wid * ROWS
    pltpu.sync_copy(features_hbm.at[pl.ds(base, ROWS)], feats_v)
    pltpu.sync_copy(labels_hbm.at[pl.ds(base, ROWS)], labels_v)

    zeros16 = jnp.zeros((16,), jnp.float32)

    def _zero(i, carry):
        for j in range(NBLK):
            sums_v[i, pl.ds(16 * j, 16)] = zeros16
        return carry

    lax.fori_loop(0, NUM_CLASSES, _zero, 0)

    def _group(g, carry):
        lab_vec = labels_v[pl.ds(16 * g, 16)]
        rows_idx = 16 * g + lax.iota(jnp.int32, 16)
        acc = zeros16
        for c in range(DIM):
            v = plsc.load_gather(feats_v, [rows_idx, jnp.full((16,), c, jnp.int32)])
            acc = acc + v * v
        inv_vec = _rsqrt16(acc)
        for k in range(16):
            r = 16 * g + k
            inv = jnp.full((16,), inv_vec[k])
            lab = lab_vec[k]
            for j in range(NBLK):
                vj = feats_v[r, pl.ds(16 * j, 16)]
                plsc.addupdate(sums_v.at[lab, pl.ds(16 * j, 16)], vj * inv)
        return carry

    lax.fori_loop(0, ROWS // 16, _group, 0)
    pltpu.sync_copy(sums_v, out_hbm.at[wid])


@functools.partial(
    pl.kernel,
    out_type=jax.ShapeDtypeStruct((NW, NUM_CLASSES, DIM), jnp.float32),
    mesh=plsc.VectorSubcoreMesh(core_axis_name="c", subcore_axis_name="s"),
    scratch_types=[
        pltpu.VMEM((ROWS, DIM), jnp.float32),
        pltpu.VMEM((ROWS,), jnp.int32),
        pltpu.VMEM((NUM_CLASSES, DIM), jnp.float32),
    ],
    compiler_params=pltpu.CompilerParams(use_tc_tiling_on_sc=False, needs_layout_passes=False),
)
def _sc_partial_sums(features_hbm, labels_hbm, out_hbm, feats_v, labels_v, sums_v):
    _sc_body(features_hbm, labels_hbm, out_hbm, feats_v, labels_v, sums_v)


def _tail_body(parts_ref, out_ref):
    sums = jnp.sum(parts_ref[...], axis=0)
    ncn = jnp.sqrt(jnp.sum(sums * sums, axis=1, keepdims=True))
    centers = sums / jnp.maximum(ncn, 1e-12)
    intra_mean = jnp.sum(sums * centers) / BATCH
    intra_loss = 1.0 - intra_mean
    csim = jnp.dot(centers, centers.T, preferred_element_type=jnp.float32)
    r = lax.broadcasted_iota(jnp.int32, (NUM_CLASSES, NUM_CLASSES), 0)
    c = lax.broadcasted_iota(jnp.int32, (NUM_CLASSES, NUM_CLASSES), 1)
    max_inter = jnp.max(jnp.where(r == c, -jnp.inf, csim))
    inter_loss = jnp.maximum(max_inter - MARGIN, 0.0)
    ratio = jnp.clip((max_inter - MARGIN) / (1.0 - MARGIN), 0.0, 1.0)
    out_ref[0, 0] = (1.0 + 2.0 * ratio) * intra_loss + 2.0 * (1.0 - ratio) * inter_loss


@jax.jit
def kernel(features, labels):
    parts = _sc_partial_sums(features, labels)
    out = pl.pallas_call(
        _tail_body,
        out_specs=pl.BlockSpec(memory_space=pltpu.SMEM),
        out_shape=jax.ShapeDtypeStruct((1, 1), jnp.float32),
    )(parts)
    return out[0, 0]


# speedup vs baseline: 1.4096x; 1.4096x over previous
import functools

import jax
import jax.numpy as jnp
from jax import lax
from jax.experimental import pallas as pl
from jax.experimental.pallas import tpu as pltpu
from jax.experimental.pallas import tpu_sc as plsc

MARGIN = 0.4
NUM_CLASSES = 64
BATCH = 4096
DIM = 256

NC = 2
NS = 16
NW = NC * NS
ROWS = BATCH // NW
NBLK = DIM // 16


def _rsqrt16(x):
    xi = plsc.bitcast(x, jnp.int32)
    yi = jnp.full((16,), 0x5F3759DF, jnp.int32) - (xi >> 1)
    y = plsc.bitcast(yi, jnp.float32)
    for _ in range(3):
        y = y * (1.5 - 0.5 * x * y * y)
    return y


def _sc_body(features_hbm, labels_hbm, out_hbm, feats_v, labels_v, sums_v):
    wid = lax.axis_index("s") * NC + lax.axis_index("c")
    base = wid * ROWS
    pltpu.sync_copy(features_hbm.at[pl.ds(base, ROWS)], feats_v)
    pltpu.sync_copy(labels_hbm.at[pl.ds(base, ROWS)], labels_v)

    zeros16 = jnp.zeros((16,), jnp.float32)

    def _zero(i, carry):
        for j in range(NBLK):
            sums_v[i, pl.ds(16 * j, 16)] = zeros16
        return carry

    lax.fori_loop(0, NUM_CLASSES, _zero, 0)

    def _group(g, carry):
        lab_vec = labels_v[pl.ds(16 * g, 16)]
        for k in range(16):
            r = 16 * g + k
            vs = [feats_v[r, pl.ds(16 * j, 16)] for j in range(NBLK)]
            sq = [vs[j] * vs[j] for j in range(NBLK)]
            while len(sq) > 1:
                sq = [sq[i] + sq[i + 1] for i in range(0, len(sq), 2)]
            t = jnp.sum(sq[0])
            inv = _rsqrt16(jnp.full((16,), t))
            lab = lab_vec[k]
            for j in range(NBLK):
                plsc.addupdate(sums_v.at[lab, pl.ds(16 * j, 16)], vs[j] * inv)
        return carry

    lax.fori_loop(0, ROWS // 16, _group, 0)
    pltpu.sync_copy(sums_v, out_hbm.at[wid])


@functools.partial(
    pl.kernel,
    out_type=jax.ShapeDtypeStruct((NW, NUM_CLASSES, DIM), jnp.float32),
    mesh=plsc.VectorSubcoreMesh(core_axis_name="c", subcore_axis_name="s"),
    scratch_types=[
        pltpu.VMEM((ROWS, DIM), jnp.float32),
        pltpu.VMEM((ROWS,), jnp.int32),
        pltpu.VMEM((NUM_CLASSES, DIM), jnp.float32),
    ],
    compiler_params=pltpu.CompilerParams(use_tc_tiling_on_sc=False, needs_layout_passes=False),
)
def _sc_partial_sums(features_hbm, labels_hbm, out_hbm, feats_v, labels_v, sums_v):
    _sc_body(features_hbm, labels_hbm, out_hbm, feats_v, labels_v, sums_v)


def _tail_body(parts_ref, out_ref):
    sums = jnp.sum(parts_ref[...], axis=0)
    ncn = jnp.sqrt(jnp.sum(sums * sums, axis=1, keepdims=True))
    centers = sums / jnp.maximum(ncn, 1e-12)
    intra_mean = jnp.sum(sums * centers) / BATCH
    intra_loss = 1.0 - intra_mean
    csim = jnp.dot(centers, centers.T, preferred_element_type=jnp.float32)
    r = lax.broadcasted_iota(jnp.int32, (NUM_CLASSES, NUM_CLASSES), 0)
    c = lax.broadcasted_iota(jnp.int32, (NUM_CLASSES, NUM_CLASSES), 1)
    max_inter = jnp.max(jnp.where(r == c, -jnp.inf, csim))
    inter_loss = jnp.maximum(max_inter - MARGIN, 0.0)
    ratio = jnp.clip((max_inter - MARGIN) / (1.0 - MARGIN), 0.0, 1.0)
    out_ref[0, 0] = (1.0 + 2.0 * ratio) * intra_loss + 2.0 * (1.0 - ratio) * inter_loss


@jax.jit
def kernel(features, labels):
    parts = _sc_partial_sums(features, labels)
    out = pl.pallas_call(
        _tail_body,
        out_specs=pl.BlockSpec(memory_space=pltpu.SMEM),
        out_shape=jax.ShapeDtypeStruct((1, 1), jnp.float32),
    )(parts)
    return out[0, 0]


# speedup vs baseline: 1.6286x; 1.1554x over previous
import functools

import jax
import jax.numpy as jnp
from jax import lax
from jax.experimental import pallas as pl
from jax.experimental.pallas import tpu as pltpu
from jax.experimental.pallas import tpu_sc as plsc

MARGIN = 0.4
NUM_CLASSES = 64
BATCH = 4096
DIM = 256

NC = 2
NS = 16
NW = NC * NS
ROWS = BATCH // NW
NBLK = DIM // 16


def _rsqrt16(x):
    xi = plsc.bitcast(x, jnp.int32)
    yi = jnp.full((16,), 0x5F3759DF, jnp.int32) - (xi >> 1)
    y = plsc.bitcast(yi, jnp.float32)
    for _ in range(3):
        y = y * (1.5 - 0.5 * x * y * y)
    return y


def _sc_body(features_hbm, labels_hbm, out_hbm, feats_v, labels_v, sums_v):
    wid = lax.axis_index("s") * NC + lax.axis_index("c")
    base = wid * ROWS
    pltpu.sync_copy(features_hbm.at[pl.ds(base, ROWS)], feats_v)
    pltpu.sync_copy(labels_hbm.at[pl.ds(base, ROWS)], labels_v)

    zeros16 = jnp.zeros((16,), jnp.float32)

    def _zero(i, carry):
        for j in range(NBLK):
            sums_v[i, pl.ds(16 * j, 16)] = zeros16
        return carry

    lax.fori_loop(0, NUM_CLASSES, _zero, 0)

    def _group(g, carry):
        lab_vec = labels_v[pl.ds(16 * g, 16)]
        for k in range(16):
            r = 16 * g + k
            vs = [feats_v[r, pl.ds(16 * j, 16)] for j in range(NBLK)]
            sq = [vs[j] * vs[j] for j in range(NBLK)]
            while len(sq) > 1:
                sq = [sq[i] + sq[i + 1] for i in range(0, len(sq), 2)]
            t = jnp.sum(sq[0])
            inv = _rsqrt16(jnp.full((16,), t))
            lab = lab_vec[k]
            for j in range(NBLK):
                plsc.addupdate(sums_v.at[lab, pl.ds(16 * j, 16)], vs[j] * inv)
        return carry

    lax.fori_loop(0, ROWS // 16, _group, 0)
    pltpu.sync_copy(sums_v, out_hbm.at[wid])


@functools.partial(
    pl.kernel,
    out_type=jax.ShapeDtypeStruct((NW, NUM_CLASSES, DIM), jnp.float32),
    mesh=plsc.VectorSubcoreMesh(core_axis_name="c", subcore_axis_name="s"),
    scratch_types=[
        pltpu.VMEM((ROWS, DIM), jnp.float32),
        pltpu.VMEM((ROWS,), jnp.int32),
        pltpu.VMEM((NUM_CLASSES, DIM), jnp.float32),
    ],
    compiler_params=pltpu.CompilerParams(use_tc_tiling_on_sc=True, needs_layout_passes=False),
)
def _sc_partial_sums(features_hbm, labels_hbm, out_hbm, feats_v, labels_v, sums_v):
    _sc_body(features_hbm, labels_hbm, out_hbm, feats_v, labels_v, sums_v)


def _tail_body(parts_ref, out_ref):
    sums = jnp.sum(parts_ref[...], axis=0)
    ncn = jnp.sqrt(jnp.sum(sums * sums, axis=1, keepdims=True))
    centers = sums / jnp.maximum(ncn, 1e-12)
    intra_mean = jnp.sum(sums * centers) / BATCH
    intra_loss = 1.0 - intra_mean
    csim = jnp.dot(centers, centers.T, preferred_element_type=jnp.float32)
    r = lax.broadcasted_iota(jnp.int32, (NUM_CLASSES, NUM_CLASSES), 0)
    c = lax.broadcasted_iota(jnp.int32, (NUM_CLASSES, NUM_CLASSES), 1)
    max_inter = jnp.max(jnp.where(r == c, -jnp.inf, csim))
    inter_loss = jnp.maximum(max_inter - MARGIN, 0.0)
    ratio = jnp.clip((max_inter - MARGIN) / (1.0 - MARGIN), 0.0, 1.0)
    out_ref[0, 0] = (1.0 + 2.0 * ratio) * intra_loss + 2.0 * (1.0 - ratio) * inter_loss


@jax.jit
def kernel(features, labels):
    parts = _sc_partial_sums(features, labels)
    out = pl.pallas_call(
        _tail_body,
        out_specs=pl.BlockSpec(memory_space=pltpu.SMEM),
        out_shape=jax.ShapeDtypeStruct((1, 1), jnp.float32),
    )(parts)
    return out[0, 0]


# speedup vs baseline: 1.6704x; 1.0256x over previous
import functools

import jax
import jax.numpy as jnp
from jax import lax
from jax.experimental import pallas as pl
from jax.experimental.pallas import tpu as pltpu
from jax.experimental.pallas import tpu_sc as plsc

MARGIN = 0.4
NUM_CLASSES = 64
BATCH = 4096
DIM = 256

NC = 2
NS = 16
NW = NC * NS
ROWS = BATCH // NW
NBLK = DIM // 16


def _rsqrt16(x):
    xi = plsc.bitcast(x, jnp.int32)
    yi = jnp.full((16,), 0x5F3759DF, jnp.int32) - (xi >> 1)
    y = plsc.bitcast(yi, jnp.float32)
    for _ in range(3):
        y = y * (1.5 - 0.5 * x * y * y)
    return y


def _sc_body(
    features_hbm, labels_hbm, out_hbm, feats_v, labels_v, sums_v, sem0, sem1
):
    wid = lax.axis_index("s") * NC + lax.axis_index("c")
    base = wid * ROWS
    half = ROWS // 2
    cp0 = pltpu.make_async_copy(
        features_hbm.at[pl.ds(base, half)], feats_v.at[pl.ds(0, half)], sem0
    )
    cp1 = pltpu.make_async_copy(
        features_hbm.at[pl.ds(base + half, half)],
        feats_v.at[pl.ds(half, half)],
        sem1,
    )
    cp0.start()
    cp1.start()
    pltpu.sync_copy(labels_hbm.at[pl.ds(base, ROWS)], labels_v)

    zeros16 = jnp.zeros((16,), jnp.float32)

    def _zero(i, carry):
        for j in range(NBLK):
            sums_v[i, pl.ds(16 * j, 16)] = zeros16
        return carry

    lax.fori_loop(0, NUM_CLASSES, _zero, 0)

    def _group(g, carry):
        lab_vec = labels_v[pl.ds(16 * g, 16)]
        rows = [16 * g + k for k in range(16)]
        accs = [zeros16] * 16
        for j in range(NBLK):
            vj = [feats_v[rows[k], pl.ds(16 * j, 16)] for k in range(16)]
            sqj = [vj[k] * vj[k] for k in range(16)]
            accs = [accs[k] + sqj[k] for k in range(16)]
        ts = [jnp.sum(accs[k]) for k in range(16)]
        tv = [jnp.full((16,), ts[k]) for k in range(16)]
        xi = [plsc.bitcast(tv[k], jnp.int32) for k in range(16)]
        magic = jnp.full((16,), 0x5F3759DF, jnp.int32)
        ys = [plsc.bitcast(magic - (xi[k] >> 1), jnp.float32) for k in range(16)]
        hs = [0.5 * tv[k] for k in range(16)]
        for _ in range(3):
            aa = [ys[k] * ys[k] for k in range(16)]
            bb = [hs[k] * aa[k] for k in range(16)]
            cc = [1.5 - bb[k] for k in range(16)]
            ys = [ys[k] * cc[k] for k in range(16)]
        labs = [lab_vec[k] for k in range(16)]
        for j in range(NBLK):
            vj = [feats_v[rows[k], pl.ds(16 * j, 16)] for k in range(16)]
            wj = [vj[k] * ys[k] for k in range(16)]
            for k in range(16):
                plsc.addupdate(sums_v.at[labs[k], pl.ds(16 * j, 16)], wj[k])
        return carry

    cp0.wait()
    lax.fori_loop(0, ROWS // 32, _group, 0)
    cp1.wait()
    lax.fori_loop(ROWS // 32, ROWS // 16, _group, 0)
    pltpu.sync_copy(sums_v, out_hbm.at[wid])


@functools.partial(
    pl.kernel,
    out_type=jax.ShapeDtypeStruct((NW, NUM_CLASSES, DIM), jnp.float32),
    mesh=plsc.VectorSubcoreMesh(core_axis_name="c", subcore_axis_name="s"),
    scratch_types=[
        pltpu.VMEM((ROWS, DIM), jnp.float32),
        pltpu.VMEM((ROWS,), jnp.int32),
        pltpu.VMEM((NUM_CLASSES, DIM), jnp.float32),
        pltpu.SemaphoreType.DMA,
        pltpu.SemaphoreType.DMA,
    ],
    compiler_params=pltpu.CompilerParams(use_tc_tiling_on_sc=True, needs_layout_passes=False),
)
def _sc_partial_sums(
    features_hbm, labels_hbm, out_hbm, feats_v, labels_v, sums_v, sem0, sem1
):
    _sc_body(features_hbm, labels_hbm, out_hbm, feats_v, labels_v, sums_v, sem0, sem1)


def _tail_body(parts_ref, out_ref):
    sums = jnp.sum(parts_ref[...], axis=0)
    ncn = jnp.sqrt(jnp.sum(sums * sums, axis=1, keepdims=True))
    centers = sums / jnp.maximum(ncn, 1e-12)
    intra_mean = jnp.sum(sums * centers) / BATCH
    intra_loss = 1.0 - intra_mean
    csim = jnp.dot(centers, centers.T, preferred_element_type=jnp.float32)
    r = lax.broadcasted_iota(jnp.int32, (NUM_CLASSES, NUM_CLASSES), 0)
    c = lax.broadcasted_iota(jnp.int32, (NUM_CLASSES, NUM_CLASSES), 1)
    max_inter = jnp.max(jnp.where(r == c, -jnp.inf, csim))
    inter_loss = jnp.maximum(max_inter - MARGIN, 0.0)
    ratio = jnp.clip((max_inter - MARGIN) / (1.0 - MARGIN), 0.0, 1.0)
    out_ref[0, 0] = (1.0 + 2.0 * ratio) * intra_loss + 2.0 * (1.0 - ratio) * inter_loss


@jax.jit
def kernel(features, labels):
    parts = _sc_partial_sums(features, labels)
    out = pl.pallas_call(
        _tail_body,
        out_specs=pl.BlockSpec(memory_space=pltpu.SMEM),
        out_shape=jax.ShapeDtypeStruct((1, 1), jnp.float32),
    )(parts)
    return out[0, 0]
